# baseline (speedup 1.0000x reference)
import jax
import jax.numpy as jnp
from jax import lax
from jax.experimental import pallas as pl
from jax.experimental.pallas import tpu as pltpu
from jax.experimental.pallas import tpu_sc as plsc

_NCORE = 2
_NSUB = 16
_NW = _NCORE * _NSUB
_B = 64
_BQ = _B // 16

_CFG = [(49, 196, 1), (196, 782, 1), (782, 3125, 1), (3125, 12500, 1),
        (12500, 50000, 3)]


def _geom(nf):
    fpt = -(-nf // (_NW * 16)) * 16
    nfp = fpt * _NW
    for fc in range(128, 0, -16):
        if fpt % fc == 0:
            return fpt, nfp, fc, fpt // fc
    raise ValueError(nf)


def _gather16(v, idx):
    dn = lax.GatherDimensionNumbers(offset_dims=(), collapsed_slice_dims=(0,),
                                    start_index_map=(0,))
    return lax.gather(v, idx[:, None], dn, slice_sizes=(1,),
                      mode=lax.GatherScatterMode.PROMISE_IN_BOUNDS)


def _bcast_lane(v, j):
    return _gather16(v, jnp.full((16,), j, dtype=jnp.int32))


def _allsum16(v, lane):
    for m in (1, 2, 4, 8):
        v = v + _gather16(v, lane ^ m)
    return v


def _rsqrt16(x):
    i = lax.bitcast_convert_type(x, jnp.int32)
    i = jnp.int32(0x5F3759DF) - (i >> 1)
    y = lax.bitcast_convert_type(i, jnp.float32)
    for _ in range(3):
        y = y * (1.5 - (0.5 * x) * y * y)
    return y


def _run_pipeline(nch, issue_g, wait_g, compute, issue_o, wait_o):
    if nch == 1:
        wait_g(0)
        compute(0, 0)
        issue_o(0, 0)
        wait_o(0)
        return
    npairs = nch // 2

    def pair(i, carry):
        a = 2 * i
        issue_g(a + 1, 1)

        @pl.when(i > 0)
        def _():
            wait_o(0)
        wait_g(0)
        compute(a, 0)
        issue_o(a, 0)

        @pl.when(a + 2 < nch)
        def _():
            issue_g(a + 2, 0)

        @pl.when(i > 0)
        def _():
            wait_o(1)
        wait_g(1)
        compute(a + 1, 1)
        issue_o(a + 1, 1)
        return carry

    lax.fori_loop(0, npairs, pair, 0)
    if nch % 2:
        t = nch - 1
        wait_o(0)
        wait_g(0)
        compute(t, 0)
        issue_o(t, 0)
    wait_o(0)
    wait_o(1)


def _make_unpool_bn(nfp, fpt, fc, nch):
    mesh = plsc.VectorSubcoreMesh(core_axis_name="c", subcore_axis_name="s",
                                  num_cores=_NCORE, num_subcores=_NSUB)
    ngrp = fc // 16

    def body(hprev, srcT, wT, gT, bT, hout, idx_v, w_v, g_v, b_v,
             rows0, rows1, out0, out1, semin, semg0, semg1, semo0, semo1):
        wid = lax.axis_index("s") * _NCORE + lax.axis_index("c")
        cps = [pltpu.async_copy(srcT.at[k, pl.ds(wid * fpt, fpt)],
                                idx_v.at[k], semin) for k in range(3)]
        cps += [pltpu.async_copy(wT.at[k, pl.ds(wid * fpt, fpt)],
                                 w_v.at[k], semin) for k in range(3)]
        cps.append(pltpu.async_copy(gT.at[pl.ds(wid * fpt, fpt)], g_v, semin))
        cps.append(pltpu.async_copy(bT.at[pl.ds(wid * fpt, fpt)], b_v, semin))
        for cp in cps[:3]:
            cp.wait()
        rows = [rows0, rows1]
        outs = [out0, out1]
        semgs = [semg0, semg1]
        semos = [semo0, semo1]
        lane = lax.iota(jnp.int32, 16)

        def issue_g(ci, p):
            for k in range(3):
                pltpu.async_copy(hprev.at[idx_v.at[k, pl.ds(ci * fc, fc)]],
                                 rows[p].at[k], semgs[p])

        def wait_g(p):
            for k in range(3):
                pltpu.make_async_copy(
                    hprev.at[idx_v.at[k, pl.ds(0, fc)]],
                    rows[p].at[k], semgs[p]).wait()

        def issue_o(ci, p):
            pltpu.async_copy(outs[p],
                             hout.at[pl.ds(wid * fpt + ci * fc, fc)], semos[p])

        def wait_o(p):
            pltpu.make_async_copy(outs[p], hout.at[pl.ds(wid * fpt, fc)],
                                  semos[p]).wait()

        def compute(ci, p):
            rv = rows[p]
            ov = outs[p]

            def group(g, c2):
                f0 = ci * fc + g * 16
                r0 = g * 16
                w16 = [w_v[k, pl.ds(f0, 16)] for k in range(3)]
                g16 = g_v[pl.ds(f0, 16)]
                b16 = b_v[pl.ds(f0, 16)]
                s1 = jnp.zeros((16,), jnp.float32)
                s2 = jnp.zeros((16,), jnp.float32)
                for j in range(16):
                    wb = [_bcast_lane(w16[k], j) for k in range(3)]
                    u = []
                    for q in range(_BQ):
                        uq = rv[0, r0 + j, pl.ds(q * 16, 16)] * wb[0]
                        uq = uq + rv[1, r0 + j, pl.ds(q * 16, 16)] * wb[1]
                        uq = uq + rv[2, r0 + j, pl.ds(q * 16, 16)] * wb[2]
                        ov[r0 + j, pl.ds(q * 16, 16)] = uq
                        u.append(uq)
                    s = (u[0] + u[1]) + (u[2] + u[3])
                    ss = (u[0] * u[0] + u[1] * u[1]) + (u[2] * u[2] + u[3] * u[3])
                    sel = lane == j
                    s1 = jnp.where(sel, _allsum16(s, lane), s1)
                    s2 = jnp.where(sel, _allsum16(ss, lane), s2)
                mean = s1 * (1.0 / _B)
                var = s2 * (1.0 / _B) - mean * mean
                rs = _rsqrt16(var + 1e-5)
                scale = g16 * rs
                shift = b16 - mean * scale
                for j in range(16):
                    sc = _bcast_lane(scale, j)
                    sh = _bcast_lane(shift, j)
                    for q in range(_BQ):
                        uq = ov[r0 + j, pl.ds(q * 16, 16)]
                        ov[r0 + j, pl.ds(q * 16, 16)] = jnp.maximum(
                            uq * sc + sh, 0.0)
                return c2

            lax.fori_loop(0, ngrp, group, 0)

        issue_g(0, 0)
        for cp in cps[3:]:
            cp.wait()
        _run_pipeline(nch, issue_g, wait_g, compute, issue_o, wait_o)

    return pl.kernel(
        body,
        out_type=jax.ShapeDtypeStruct((nfp, _B), jnp.float32),
        mesh=mesh,
        compiler_params=pltpu.CompilerParams(use_tc_tiling_on_sc=False, needs_layout_passes=False),
        scratch_types=[
            pltpu.VMEM((3, fpt), jnp.int32),
            pltpu.VMEM((3, fpt), jnp.float32),
            pltpu.VMEM((fpt,), jnp.float32),
            pltpu.VMEM((fpt,), jnp.float32),
            pltpu.VMEM((3, fc, _B), jnp.float32),
            pltpu.VMEM((3, fc, _B), jnp.float32),
            pltpu.VMEM((fc, _B), jnp.float32),
            pltpu.VMEM((fc, _B), jnp.float32),
            pltpu.SemaphoreType.DMA,
            pltpu.SemaphoreType.DMA,
            pltpu.SemaphoreType.DMA,
            pltpu.SemaphoreType.DMA,
            pltpu.SemaphoreType.DMA,
        ],
    )


def _make_unpool_final(nfp, fpt, fc, nch, cout):
    mesh = plsc.VectorSubcoreMesh(core_axis_name="c", subcore_axis_name="s",
                                  num_cores=_NCORE, num_subcores=_NSUB)
    ngrp = fc // 16
    fcp = fc + 1

    def body(hprev, srcT, wT, hout, idx_v, w_v,
             rows0, rows1, out0, out1, semin, semg0, semg1, semo0, semo1):
        wid = lax.axis_index("s") * _NCORE + lax.axis_index("c")
        cps = [pltpu.async_copy(srcT.at[k, pl.ds(wid * fpt, fpt)],
                                idx_v.at[k], semin) for k in range(3)]
        cps += [pltpu.async_copy(wT.at[k, pl.ds(wid * fpt * cout, fpt * cout)],
                                 w_v.at[k], semin) for k in range(3)]
        for cp in cps[:3]:
            cp.wait()
        rows = [rows0, rows1]
        outs = [out0, out1]
        semgs = [semg0, semg1]
        semos = [semo0, semo1]
        lane = lax.iota(jnp.int32, 16)
        lane3 = lane * cout

        def issue_g(ci, p):
            for k in range(3):
                pltpu.async_copy(hprev.at[idx_v.at[k, pl.ds(ci * fc, fc)]],
                                 rows[p].at[k], semgs[p])

        def wait_g(p):
            for k in range(3):
                pltpu.make_async_copy(
                    hprev.at[idx_v.at[k, pl.ds(0, fc)]],
                    rows[p].at[k], semgs[p]).wait()

        def issue_o(ci, p):
            pltpu.async_copy(
                outs[p].at[:, :, pl.ds(0, fc)],
                hout.at[:, :, pl.ds(wid * fpt + ci * fc, fc)], semos[p])

        def wait_o(p):
            pltpu.make_async_copy(
                outs[p].at[:, :, pl.ds(0, fc)],
                hout.at[:, :, pl.ds(wid * fpt, fc)], semos[p]).wait()

        def compute(ci, p):
            rv = rows[p]
            ov = outs[p]

            def group(g, c2):
                f0 = ci * fc + g * 16
                r0 = g * 16
                w16 = [[plsc.load_gather(
                            w_v, [jnp.full((16,), k, dtype=jnp.int32),
                                  lane3 + (f0 * cout + c)])
                        for c in range(cout)] for k in range(3)]
                for j in range(16):
                    wb = [[_bcast_lane(w16[k][c], j) for c in range(cout)]
                          for k in range(3)]
                    r = [[rv[k, r0 + j, pl.ds(q * 16, 16)]
                          for q in range(_BQ)] for k in range(3)]
                    fj = jnp.full((16,), r0 + j, dtype=jnp.int32)
                    for c in range(cout):
                        cc = jnp.full((16,), c, dtype=jnp.int32)
                        for q in range(_BQ):
                            val = (r[0][q] * wb[0][c] + r[1][q] * wb[1][c]
                                   + r[2][q] * wb[2][c])
                            plsc.store_scatter(ov, [cc, lane + (q * 16), fj],
                                               val)
                return c2

            lax.fori_loop(0, ngrp, group, 0)

        issue_g(0, 0)
        for cp in cps[3:]:
            cp.wait()
        _run_pipeline(nch, issue_g, wait_g, compute, issue_o, wait_o)

    return pl.kernel(
        body,
        out_type=jax.ShapeDtypeStruct((cout, _B, nfp), jnp.float32),
        mesh=mesh,
        compiler_params=pltpu.CompilerParams(use_tc_tiling_on_sc=False, needs_layout_passes=False),
        scratch_types=[
            pltpu.VMEM((3, fpt), jnp.int32),
            pltpu.VMEM((3, fpt * cout), jnp.float32),
            pltpu.VMEM((3, fc, _B), jnp.float32),
            pltpu.VMEM((3, fc, _B), jnp.float32),
            pltpu.VMEM((cout, _B, fcp), jnp.float32),
            pltpu.VMEM((cout, _B, fcp), jnp.float32),
            pltpu.SemaphoreType.DMA,
            pltpu.SemaphoreType.DMA,
            pltpu.SemaphoreType.DMA,
            pltpu.SemaphoreType.DMA,
            pltpu.SemaphoreType.DMA,
        ],
    )


def _build_levels():
    ks = []
    for cfg in _CFG[:4]:
        fpt, nfp, fc, nch = _geom(cfg[1])
        ks.append(_make_unpool_bn(nfp, fpt, fc, nch))
    fpt, nfp, fc, nch = _geom(_CFG[4][1])
    ks.append(_make_unpool_final(nfp, fpt, fc, nch, _CFG[4][2]))
    return ks


_LEVELS_CACHE = []


def _levels():
    if not _LEVELS_CACHE:
        _LEVELS_CACHE.extend(_build_levels())
    return _LEVELS_CACHE

def _eps():
    return [jax.random.normal(jax.random.key(100 + i),
                              (3 * _CFG[i][1] * _CFG[i][2],),
                              dtype=jnp.float32)
            for i in range(5)]


def _make_prep(nlev):
    def prep_body(*refs):
        mus, rhos, epss = refs[0:nlev], refs[nlev:2 * nlev], refs[2 * nlev:3 * nlev]
        ws, kl_ref = refs[3 * nlev:4 * nlev], refs[4 * nlev]
        kl = jnp.zeros((), jnp.float32)
        for i in range(nlev):
            mu = mus[i][...]
            rho = rhos[i][...]
            sig = jnp.log1p(jnp.exp(rho))
            ws[i][...] = mu + sig * epss[i][...]
            kl = kl + jnp.sum(0.5 * (mu * mu + sig * sig)
                              - jnp.log(sig + 1e-12) - 0.5)
        kl_ref[...] = kl.reshape(1, 1)
    return prep_body


_PREP14 = pl.pallas_call(
    _make_prep(4),
    out_shape=tuple(jax.ShapeDtypeStruct((3 * cfg[1] * cfg[2],), jnp.float32)
                    for cfg in _CFG[:4]) + (jax.ShapeDtypeStruct((1, 1),
                                                                 jnp.float32),),
)
_PREP5 = pl.pallas_call(
    _make_prep(1),
    out_shape=(jax.ShapeDtypeStruct((3 * _CFG[4][1] * _CFG[4][2],),
                                    jnp.float32),
               jax.ShapeDtypeStruct((1, 1), jnp.float32)),
)


def kernel(x, idx45, idx34, idx23, idx12, idx01,
           w_mu1, w_rho1, w_mu2, w_rho2, w_mu3, w_rho3, w_mu4, w_rho4,
           w_mu5, w_rho5, g1, b1, g2, b2, g3, b3, g4, b4):
    idxs = [idx45, idx34, idx23, idx12, idx01]
    mus = [w_mu1, w_mu2, w_mu3, w_mu4, w_mu5]
    rhos = [w_rho1, w_rho2, w_rho3, w_rho4, w_rho5]
    gs = [g1, g2, g3, g4]
    bs = [b1, b2, b3, b4]

    eps = _eps()
    p5 = _PREP5(mus[4].reshape(-1), rhos[4].reshape(-1), eps[4])
    p14 = _PREP14(*[m.reshape(-1) for m in mus[:4]],
                  *[r.reshape(-1) for r in rhos[:4]], *eps[:4])
    w_list = list(p14[:4]) + [p5[0]]
    kl = p14[4] + p5[1]

    lv = _levels()
    h = x.T
    out5 = None
    for i, (ncoarse, nf, cout) in enumerate(_CFG):
        fpt, nfp, fcw, nch = _geom(nf)
        w = w_list[i]
        wT = jnp.pad(w.reshape(3, nf * cout), ((0, 0), (0, (nfp - nf) * cout)))
        srcT = jnp.pad(idxs[i][0].reshape(3, nf), ((0, 0), (0, nfp - nf)))
        if i < 4:
            gT = jnp.pad(gs[i], (0, nfp - nf))
            bT = jnp.pad(bs[i], (0, nfp - nf))
            h = lv[i](h, srcT, wT, gT, bT)
        else:
            out5 = lv[4](h, srcT, wT)

    out = out5.transpose(1, 2, 0)[:, :_CFG[4][1], :]
    return out, kl[0, 0]


# speedup vs baseline: 1.1368x; 1.1368x over previous
import jax
import jax.numpy as jnp
from jax import lax
from jax.experimental import pallas as pl
from jax.experimental.pallas import tpu as pltpu
from jax.experimental.pallas import tpu_sc as plsc

_NCORE = 2
_NSUB = 16
_NW = _NCORE * _NSUB
_B = 64
_BQ = _B // 16

_CFG = [(49, 196, 1), (196, 782, 1), (782, 3125, 1), (3125, 12500, 1),
        (12500, 50000, 3)]


def _geom(nf):
    fpt = -(-nf // (_NW * 16)) * 16
    nfp = fpt * _NW
    for fc in range(128, 0, -16):
        if fpt % fc == 0:
            return fpt, nfp, fc, fpt // fc
    raise ValueError(nf)


def _gather16(v, idx):
    dn = lax.GatherDimensionNumbers(offset_dims=(), collapsed_slice_dims=(0,),
                                    start_index_map=(0,))
    return lax.gather(v, idx[:, None], dn, slice_sizes=(1,),
                      mode=lax.GatherScatterMode.PROMISE_IN_BOUNDS)


def _bcast_lane(v, j):
    return _gather16(v, jnp.full((16,), j, dtype=jnp.int32))


def _allsum16(v, lane):
    for m in (1, 2, 4, 8):
        v = v + _gather16(v, lane ^ m)
    return v


def _rsqrt16(x):
    i = lax.bitcast_convert_type(x, jnp.int32)
    i = jnp.int32(0x5F3759DF) - (i >> 1)
    y = lax.bitcast_convert_type(i, jnp.float32)
    for _ in range(3):
        y = y * (1.5 - (0.5 * x) * y * y)
    return y


def _run_pipeline(nch, issue_g, wait_g, compute, issue_o, wait_o):
    if nch == 1:
        wait_g(0)
        compute(0, 0)
        issue_o(0, 0)
        wait_o(0)
        return
    npairs = nch // 2

    def pair(i, carry):
        a = 2 * i
        issue_g(a + 1, 1)

        @pl.when(i > 0)
        def _():
            wait_o(0)
        wait_g(0)
        compute(a, 0)
        issue_o(a, 0)

        @pl.when(a + 2 < nch)
        def _():
            issue_g(a + 2, 0)

        @pl.when(i > 0)
        def _():
            wait_o(1)
        wait_g(1)
        compute(a + 1, 1)
        issue_o(a + 1, 1)
        return carry

    lax.fori_loop(0, npairs, pair, 0)
    if nch % 2:
        t = nch - 1
        wait_o(0)
        wait_g(0)
        compute(t, 0)
        issue_o(t, 0)
    wait_o(0)
    wait_o(1)


def _geom1(nf, nsub=16):
    base = -(-nf // (nsub * 16)) * 16
    for fpt in range(base, base + 9 * 16, 16):
        for fc in (112, 96, 80, 64, 48, 32, 16):
            if fc <= fpt and fpt % fc == 0:
                return fpt, fpt * nsub, fc, fpt // fc
    raise ValueError(nf)


_G1 = [_geom1(cfg[1]) for cfg in _CFG[:4]]
_FPTMAX = max(g[0] for g in _G1)
_FCMAX = max(g[2] for g in _G1)


def _make_unpool_fused14():
    mesh = plsc.VectorSubcoreMesh(core_axis_name="c", subcore_axis_name="s",
                                  num_cores=1, num_subcores=_NSUB)

    def body(*refs):
        (x_t, s1, s2, s3, s4, w1, w2, w3, w4, gg1, gg2, gg3, gg4,
         bb1, bb2, bb3, bb4) = refs[:17]
        houts = refs[17:21]
        (idx_v, w_v, g_v, b_v, rows0, rows1, out0, out1,
         semin, semg0, semg1, semo0, semo1) = refs[21:]
        srcs = [s1, s2, s3, s4]
        wts = [w1, w2, w3, w4]
        gts = [gg1, gg2, gg3, gg4]
        bts = [bb1, bb2, bb3, bb4]
        wid = lax.axis_index("s")
        rows = [rows0, rows1]
        outs = [out0, out1]
        semgs = [semg0, semg1]
        semos = [semo0, semo1]
        lane = lax.iota(jnp.int32, 16)
        hsrcs = [x_t] + list(houts[:3])

        for li in range(4):
            fpt, _, fc, nch = _G1[li]
            gu = 16 if nch == 1 else 8
            ngrp = fc // gu
            hprev, hout = hsrcs[li], houts[li]
            srcT, wT, gT, bT = srcs[li], wts[li], gts[li], bts[li]
            cps = [pltpu.async_copy(srcT.at[k, pl.ds(wid * fpt, fpt)],
                                    idx_v.at[k, pl.ds(0, fpt)], semin)
                   for k in range(3)]
            cps += [pltpu.async_copy(wT.at[k, pl.ds(wid * fpt, fpt)],
                                     w_v.at[k, pl.ds(0, fpt)], semin)
                    for k in range(3)]
            cps.append(pltpu.async_copy(gT.at[pl.ds(wid * fpt, fpt)],
                                        g_v.at[pl.ds(0, fpt)], semin))
            cps.append(pltpu.async_copy(bT.at[pl.ds(wid * fpt, fpt)],
                                        b_v.at[pl.ds(0, fpt)], semin))
            for cp in cps[:3]:
                cp.wait()

            def issue_g(ci, p, fc=fc):
                for k in range(3):
                    pltpu.async_copy(
                        hprev.at[idx_v.at[k, pl.ds(ci * fc, fc)]],
                        rows[p].at[k, pl.ds(0, fc)], semgs[p])

            def wait_g(p, fc=fc):
                for k in range(3):
                    pltpu.make_async_copy(
                        hprev.at[idx_v.at[k, pl.ds(0, fc)]],
                        rows[p].at[k, pl.ds(0, fc)], semgs[p]).wait()

            def issue_o(ci, p, fpt=fpt, fc=fc):
                pltpu.async_copy(outs[p].at[pl.ds(0, fc)],
                                 hout.at[pl.ds(wid * fpt + ci * fc, fc)],
                                 semos[p])

            def wait_o(p, fpt=fpt, fc=fc):
                pltpu.make_async_copy(outs[p].at[pl.ds(0, fc)],
                                      hout.at[pl.ds(wid * fpt, fc)],
                                      semos[p]).wait()

            def compute(ci, p, fc=fc, gu=gu, ngrp=ngrp):
                rv = rows[p]
                ov = outs[p]

                def group(g, c2):
                    f0 = ci * fc + g * gu
                    r0 = g * gu
                    w16 = [w_v[k, pl.ds(f0, 16)] for k in range(3)]
                    g16 = g_v[pl.ds(f0, 16)]
                    b16 = b_v[pl.ds(f0, 16)]
                    s1v = jnp.zeros((16,), jnp.float32)
                    s2v = jnp.zeros((16,), jnp.float32)
                    for j in range(gu):
                        wb = [_bcast_lane(w16[k], j) for k in range(3)]
                        u = []
                        for q in range(_BQ):
                            uq = rv[0, r0 + j, pl.ds(q * 16, 16)] * wb[0]
                            uq = uq + rv[1, r0 + j, pl.ds(q * 16, 16)] * wb[1]
                            uq = uq + rv[2, r0 + j, pl.ds(q * 16, 16)] * wb[2]
                            ov[r0 + j, pl.ds(q * 16, 16)] = uq
                            u.append(uq)
                        s = (u[0] + u[1]) + (u[2] + u[3])
                        ss = (u[0] * u[0] + u[1] * u[1]) + (u[2] * u[2]
                                                           + u[3] * u[3])
                        sel = lane == j
                        s1v = jnp.where(sel, _allsum16(s, lane), s1v)
                        s2v = jnp.where(sel, _allsum16(ss, lane), s2v)
                    mean = s1v * (1.0 / _B)
                    var = s2v * (1.0 / _B) - mean * mean
                    rs = _rsqrt16(var + 1e-5)
                    scale = g16 * rs
                    shift = b16 - mean * scale
                    for j in range(gu):
                        sc = _bcast_lane(scale, j)
                        sh = _bcast_lane(shift, j)
                        for q in range(_BQ):
                            uq = ov[r0 + j, pl.ds(q * 16, 16)]
                            ov[r0 + j, pl.ds(q * 16, 16)] = jnp.maximum(
                                uq * sc + sh, 0.0)
                    return c2

                lax.fori_loop(0, ngrp, group, 0)

            issue_g(0, 0)
            for cp in cps[3:]:
                cp.wait()
            _run_pipeline(nch, issue_g, wait_g, compute, issue_o, wait_o)
            if li < 3:
                plsc.subcore_barrier()

    return pl.kernel(
        body,
        out_type=tuple(jax.ShapeDtypeStruct((g[1], _B), jnp.float32)
                       for g in _G1),
        mesh=mesh,
        compiler_params=pltpu.CompilerParams(use_tc_tiling_on_sc=False,
                                             needs_layout_passes=False),
        scratch_types=[
            pltpu.VMEM((3, _FPTMAX + 16), jnp.int32),
            pltpu.VMEM((3, _FPTMAX + 16), jnp.float32),
            pltpu.VMEM((_FPTMAX + 16,), jnp.float32),
            pltpu.VMEM((_FPTMAX + 16,), jnp.float32),
            pltpu.VMEM((3, _FCMAX, _B), jnp.float32),
            pltpu.VMEM((3, _FCMAX, _B), jnp.float32),
            pltpu.VMEM((_FCMAX, _B), jnp.float32),
            pltpu.VMEM((_FCMAX, _B), jnp.float32),
            pltpu.SemaphoreType.DMA,
            pltpu.SemaphoreType.DMA,
            pltpu.SemaphoreType.DMA,
            pltpu.SemaphoreType.DMA,
            pltpu.SemaphoreType.DMA,
        ],
    )


def _make_unpool_bn(nfp, fpt, fc, nch):
    mesh = plsc.VectorSubcoreMesh(core_axis_name="c", subcore_axis_name="s",
                                  num_cores=_NCORE, num_subcores=_NSUB)
    ngrp = fc // 16

    def body(hprev, srcT, wT, gT, bT, hout, idx_v, w_v, g_v, b_v,
             rows0, rows1, out0, out1, semin, semg0, semg1, semo0, semo1):
        wid = lax.axis_index("s") * _NCORE + lax.axis_index("c")
        cps = [pltpu.async_copy(srcT.at[k, pl.ds(wid * fpt, fpt)],
                                idx_v.at[k], semin) for k in range(3)]
        cps += [pltpu.async_copy(wT.at[k, pl.ds(wid * fpt, fpt)],
                                 w_v.at[k], semin) for k in range(3)]
        cps.append(pltpu.async_copy(gT.at[pl.ds(wid * fpt, fpt)], g_v, semin))
        cps.append(pltpu.async_copy(bT.at[pl.ds(wid * fpt, fpt)], b_v, semin))
        for cp in cps[:3]:
            cp.wait()
        rows = [rows0, rows1]
        outs = [out0, out1]
        semgs = [semg0, semg1]
        semos = [semo0, semo1]
        lane = lax.iota(jnp.int32, 16)

        def issue_g(ci, p):
            for k in range(3):
                pltpu.async_copy(hprev.at[idx_v.at[k, pl.ds(ci * fc, fc)]],
                                 rows[p].at[k], semgs[p])

        def wait_g(p):
            for k in range(3):
                pltpu.make_async_copy(
                    hprev.at[idx_v.at[k, pl.ds(0, fc)]],
                    rows[p].at[k], semgs[p]).wait()

        def issue_o(ci, p):
            pltpu.async_copy(outs[p],
                             hout.at[pl.ds(wid * fpt + ci * fc, fc)], semos[p])

        def wait_o(p):
            pltpu.make_async_copy(outs[p], hout.at[pl.ds(wid * fpt, fc)],
                                  semos[p]).wait()

        def compute(ci, p):
            rv = rows[p]
            ov = outs[p]

            def group(g, c2):
                f0 = ci * fc + g * 16
                r0 = g * 16
                w16 = [w_v[k, pl.ds(f0, 16)] for k in range(3)]
                g16 = g_v[pl.ds(f0, 16)]
                b16 = b_v[pl.ds(f0, 16)]
                s1 = jnp.zeros((16,), jnp.float32)
                s2 = jnp.zeros((16,), jnp.float32)
                for j in range(16):
                    wb = [_bcast_lane(w16[k], j) for k in range(3)]
                    u = []
                    for q in range(_BQ):
                        uq = rv[0, r0 + j, pl.ds(q * 16, 16)] * wb[0]
                        uq = uq + rv[1, r0 + j, pl.ds(q * 16, 16)] * wb[1]
                        uq = uq + rv[2, r0 + j, pl.ds(q * 16, 16)] * wb[2]
                        ov[r0 + j, pl.ds(q * 16, 16)] = uq
                        u.append(uq)
                    s = (u[0] + u[1]) + (u[2] + u[3])
                    ss = (u[0] * u[0] + u[1] * u[1]) + (u[2] * u[2] + u[3] * u[3])
                    sel = lane == j
                    s1 = jnp.where(sel, _allsum16(s, lane), s1)
                    s2 = jnp.where(sel, _allsum16(ss, lane), s2)
                mean = s1 * (1.0 / _B)
                var = s2 * (1.0 / _B) - mean * mean
                rs = _rsqrt16(var + 1e-5)
                scale = g16 * rs
                shift = b16 - mean * scale
                for j in range(16):
                    sc = _bcast_lane(scale, j)
                    sh = _bcast_lane(shift, j)
                    for q in range(_BQ):
                        uq = ov[r0 + j, pl.ds(q * 16, 16)]
                        ov[r0 + j, pl.ds(q * 16, 16)] = jnp.maximum(
                            uq * sc + sh, 0.0)
                return c2

            lax.fori_loop(0, ngrp, group, 0)

        issue_g(0, 0)
        for cp in cps[3:]:
            cp.wait()
        _run_pipeline(nch, issue_g, wait_g, compute, issue_o, wait_o)

    return pl.kernel(
        body,
        out_type=jax.ShapeDtypeStruct((nfp, _B), jnp.float32),
        mesh=mesh,
        compiler_params=pltpu.CompilerParams(use_tc_tiling_on_sc=False, needs_layout_passes=False),
        scratch_types=[
            pltpu.VMEM((3, fpt), jnp.int32),
            pltpu.VMEM((3, fpt), jnp.float32),
            pltpu.VMEM((fpt,), jnp.float32),
            pltpu.VMEM((fpt,), jnp.float32),
            pltpu.VMEM((3, fc, _B), jnp.float32),
            pltpu.VMEM((3, fc, _B), jnp.float32),
            pltpu.VMEM((fc, _B), jnp.float32),
            pltpu.VMEM((fc, _B), jnp.float32),
            pltpu.SemaphoreType.DMA,
            pltpu.SemaphoreType.DMA,
            pltpu.SemaphoreType.DMA,
            pltpu.SemaphoreType.DMA,
            pltpu.SemaphoreType.DMA,
        ],
    )


def _make_unpool_final(nfp, fpt, fc, nch, cout):
    mesh = plsc.VectorSubcoreMesh(core_axis_name="c", subcore_axis_name="s",
                                  num_cores=_NCORE, num_subcores=_NSUB)
    ngrp = fc // 16
    fcp = fc + 1

    def body(hprev, srcT, wT, hout, idx_v, w_v,
             rows0, rows1, out0, out1, semin, semg0, semg1, semo0, semo1):
        wid = lax.axis_index("s") * _NCORE + lax.axis_index("c")
        cps = [pltpu.async_copy(srcT.at[k, pl.ds(wid * fpt, fpt)],
                                idx_v.at[k], semin) for k in range(3)]
        cps += [pltpu.async_copy(wT.at[k, pl.ds(wid * fpt * cout, fpt * cout)],
                                 w_v.at[k], semin) for k in range(3)]
        for cp in cps[:3]:
            cp.wait()
        rows = [rows0, rows1]
        outs = [out0, out1]
        semgs = [semg0, semg1]
        semos = [semo0, semo1]
        lane = lax.iota(jnp.int32, 16)
        lane3 = lane * cout

        def issue_g(ci, p):
            for k in range(3):
                pltpu.async_copy(hprev.at[idx_v.at[k, pl.ds(ci * fc, fc)]],
                                 rows[p].at[k], semgs[p])

        def wait_g(p):
            for k in range(3):
                pltpu.make_async_copy(
                    hprev.at[idx_v.at[k, pl.ds(0, fc)]],
                    rows[p].at[k], semgs[p]).wait()

        def issue_o(ci, p):
            pltpu.async_copy(
                outs[p].at[:, :, pl.ds(0, fc)],
                hout.at[:, :, pl.ds(wid * fpt + ci * fc, fc)], semos[p])

        def wait_o(p):
            pltpu.make_async_copy(
                outs[p].at[:, :, pl.ds(0, fc)],
                hout.at[:, :, pl.ds(wid * fpt, fc)], semos[p]).wait()

        def compute(ci, p):
            rv = rows[p]
            ov = outs[p]

            def group(g, c2):
                f0 = ci * fc + g * 16
                r0 = g * 16
                w16 = [[plsc.load_gather(
                            w_v, [jnp.full((16,), k, dtype=jnp.int32),
                                  lane3 + (f0 * cout + c)])
                        for c in range(cout)] for k in range(3)]
                for j in range(16):
                    wb = [[_bcast_lane(w16[k][c], j) for c in range(cout)]
                          for k in range(3)]
                    r = [[rv[k, r0 + j, pl.ds(q * 16, 16)]
                          for q in range(_BQ)] for k in range(3)]
                    fj = jnp.full((16,), r0 + j, dtype=jnp.int32)
                    for c in range(cout):
                        cc = jnp.full((16,), c, dtype=jnp.int32)
                        for q in range(_BQ):
                            val = (r[0][q] * wb[0][c] + r[1][q] * wb[1][c]
                                   + r[2][q] * wb[2][c])
                            plsc.store_scatter(ov, [cc, lane + (q * 16), fj],
                                               val)
                return c2

            lax.fori_loop(0, ngrp, group, 0)

        issue_g(0, 0)
        for cp in cps[3:]:
            cp.wait()
        _run_pipeline(nch, issue_g, wait_g, compute, issue_o, wait_o)

    return pl.kernel(
        body,
        out_type=jax.ShapeDtypeStruct((cout, _B, nfp), jnp.float32),
        mesh=mesh,
        compiler_params=pltpu.CompilerParams(use_tc_tiling_on_sc=False, needs_layout_passes=False),
        scratch_types=[
            pltpu.VMEM((3, fpt), jnp.int32),
            pltpu.VMEM((3, fpt * cout), jnp.float32),
            pltpu.VMEM((3, fc, _B), jnp.float32),
            pltpu.VMEM((3, fc, _B), jnp.float32),
            pltpu.VMEM((cout, _B, fcp), jnp.float32),
            pltpu.VMEM((cout, _B, fcp), jnp.float32),
            pltpu.SemaphoreType.DMA,
            pltpu.SemaphoreType.DMA,
            pltpu.SemaphoreType.DMA,
            pltpu.SemaphoreType.DMA,
            pltpu.SemaphoreType.DMA,
        ],
    )


def _build_levels():
    fpt, nfp, fc, nch = _geom(_CFG[4][1])
    return [_make_unpool_fused14(),
            _make_unpool_final(nfp, fpt, fc, nch, _CFG[4][2])]


_LEVELS_CACHE = []


def _levels():
    if not _LEVELS_CACHE:
        _LEVELS_CACHE.extend(_build_levels())
    return _LEVELS_CACHE

def _eps():
    return [jax.random.normal(jax.random.key(100 + i),
                              (3 * _CFG[i][1] * _CFG[i][2],),
                              dtype=jnp.float32)
            for i in range(5)]


def _make_prep(nlev):
    def prep_body(*refs):
        mus, rhos, epss = refs[0:nlev], refs[nlev:2 * nlev], refs[2 * nlev:3 * nlev]
        ws, kl_ref = refs[3 * nlev:4 * nlev], refs[4 * nlev]
        kl = jnp.zeros((), jnp.float32)
        for i in range(nlev):
            mu = mus[i][...]
            rho = rhos[i][...]
            sig = jnp.log1p(jnp.exp(rho))
            ws[i][...] = mu + sig * epss[i][...]
            kl = kl + jnp.sum(0.5 * (mu * mu + sig * sig)
                              - jnp.log(sig + 1e-12) - 0.5)
        kl_ref[...] = kl.reshape(1, 1)
    return prep_body


_PREP14 = pl.pallas_call(
    _make_prep(4),
    out_shape=tuple(jax.ShapeDtypeStruct((3 * cfg[1] * cfg[2],), jnp.float32)
                    for cfg in _CFG[:4]) + (jax.ShapeDtypeStruct((1, 1),
                                                                 jnp.float32),),
)
_PREP5 = pl.pallas_call(
    _make_prep(1),
    out_shape=(jax.ShapeDtypeStruct((3 * _CFG[4][1] * _CFG[4][2],),
                                    jnp.float32),
               jax.ShapeDtypeStruct((1, 1), jnp.float32)),
)


def kernel(x, idx45, idx34, idx23, idx12, idx01,
           w_mu1, w_rho1, w_mu2, w_rho2, w_mu3, w_rho3, w_mu4, w_rho4,
           w_mu5, w_rho5, g1, b1, g2, b2, g3, b3, g4, b4):
    idxs = [idx45, idx34, idx23, idx12, idx01]
    mus = [w_mu1, w_mu2, w_mu3, w_mu4, w_mu5]
    rhos = [w_rho1, w_rho2, w_rho3, w_rho4, w_rho5]
    gs = [g1, g2, g3, g4]
    bs = [b1, b2, b3, b4]

    eps = _eps()
    p5 = _PREP5(mus[4].reshape(-1), rhos[4].reshape(-1), eps[4])
    p14 = _PREP14(*[m.reshape(-1) for m in mus[:4]],
                  *[r.reshape(-1) for r in rhos[:4]], *eps[:4])
    w_list = list(p14[:4]) + [p5[0]]
    kl = p14[4] + p5[1]

    lv = _levels()
    srcs, wts, gts, bts = [], [], [], []
    for i in range(4):
        nf = _CFG[i][1]
        nfp = _G1[i][1]
        wts.append(jnp.pad(w_list[i].reshape(3, nf), ((0, 0), (0, nfp - nf))))
        srcs.append(jnp.pad(idxs[i][0].reshape(3, nf),
                            ((0, 0), (0, nfp - nf))))
        gts.append(jnp.pad(gs[i], (0, nfp - nf)))
        bts.append(jnp.pad(bs[i], (0, nfp - nf)))
    hs = lv[0](x.T, *srcs, *wts, *gts, *bts)
    h4 = hs[3]

    nf = _CFG[4][1]
    cout = _CFG[4][2]
    fpt, nfp, fcw, nch = _geom(nf)
    wT = jnp.pad(w_list[4].reshape(3, nf * cout),
                 ((0, 0), (0, (nfp - nf) * cout)))
    srcT = jnp.pad(idx01[0].reshape(3, nf), ((0, 0), (0, nfp - nf)))
    out5 = lv[1](h4, srcT, wT)

    out = out5.transpose(1, 2, 0)[:, :nf, :]
    return out, kl[0, 0]
